# baseline (speedup 1.0000x reference)
import jax
import jax.numpy as jnp
from jax.experimental import pallas as pl
from jax.experimental.pallas import tpu as pltpu

QUEUE = 65536
FEAT = 128
BATCH = 4096
CHUNK = 2048
NCH = QUEUE // CHUNK


def _queue_body(ptr_smem, keys, data, out, ptr_out, sem):
    praw = ptr_smem[0]
    pc = jnp.clip(praw, 0, QUEUE - BATCH)

    for i in range(NCH):
        g = i * CHUNK
        in_keys = jnp.logical_and(g >= pc, g < pc + BATCH)

        @pl.when(in_keys)
        def _():
            pltpu.make_async_copy(
                keys.at[pl.ds(pl.multiple_of(g - pc, 8), CHUNK)],
                out.at[pl.ds(g, CHUNK)], sem).start()

        @pl.when(jnp.logical_not(in_keys))
        def _():
            pltpu.make_async_copy(
                data.at[pl.ds(g, CHUNK)],
                out.at[pl.ds(g, CHUNK)], sem).start()

    ptr_out[0] = (praw + BATCH) % QUEUE

    for i in range(NCH):
        pltpu.make_async_copy(
            data.at[pl.ds(0, CHUNK)], out.at[pl.ds(0, CHUNK)], sem).wait()


def kernel(keys, data, ptr):
    grid_spec = pltpu.PrefetchScalarGridSpec(
        num_scalar_prefetch=1,
        grid=(1,),
        in_specs=[
            pl.BlockSpec(memory_space=pl.ANY),
            pl.BlockSpec(memory_space=pl.ANY),
        ],
        out_specs=[
            pl.BlockSpec(memory_space=pl.ANY),
            pl.BlockSpec(memory_space=pltpu.SMEM),
        ],
        scratch_shapes=[pltpu.SemaphoreType.DMA],
    )
    out, new_ptr = pl.pallas_call(
        _queue_body,
        grid_spec=grid_spec,
        out_shape=(
            jax.ShapeDtypeStruct((QUEUE, FEAT), jnp.float32),
            jax.ShapeDtypeStruct((1,), jnp.int32),
        ),
    )(ptr, keys, data)
    return out, new_ptr


# speedup vs baseline: 38.2461x; 38.2461x over previous
import jax
import jax.numpy as jnp
from jax.experimental import pallas as pl
from jax.experimental.pallas import tpu as pltpu

QUEUE = 65536
FEAT = 128
BATCH = 4096
BLK = 4096
NCH = QUEUE // BLK


def _queue_body(ptr_smem, keys, data, out, ptr_out):
    i = pl.program_id(0)
    praw = ptr_smem[0]
    pc = jnp.clip(praw, 0, QUEUE - BATCH)
    g = i * BLK
    in_keys = jnp.logical_and(g >= pc, g < pc + BATCH)

    @pl.when(in_keys)
    def _():
        out[...] = keys[...]

    @pl.when(jnp.logical_not(in_keys))
    def _():
        out[...] = data[...]

    ptr_out[0] = (praw + BATCH) % QUEUE


def kernel(keys, data, ptr):
    grid_spec = pltpu.PrefetchScalarGridSpec(
        num_scalar_prefetch=1,
        grid=(NCH,),
        in_specs=[
            pl.BlockSpec((BATCH, FEAT), lambda i, p: (0, 0)),
            pl.BlockSpec((BLK, FEAT), lambda i, p: (i, 0)),
        ],
        out_specs=[
            pl.BlockSpec((BLK, FEAT), lambda i, p: (i, 0)),
            pl.BlockSpec(memory_space=pltpu.SMEM),
        ],
    )
    out, new_ptr = pl.pallas_call(
        _queue_body,
        grid_spec=grid_spec,
        out_shape=(
            jax.ShapeDtypeStruct((QUEUE, FEAT), jnp.float32),
            jax.ShapeDtypeStruct((1,), jnp.int32),
        ),
    )(ptr, keys, data)
    return out, new_ptr


# speedup vs baseline: 38.5327x; 1.0075x over previous
import jax
import jax.numpy as jnp
from jax.experimental import pallas as pl
from jax.experimental.pallas import tpu as pltpu

QUEUE = 65536
FEAT = 128
BATCH = 4096
BLK = 4096
NCH = QUEUE // BLK


def _queue_body(ptr_smem, keys, data, out, ptr_out):
    i = pl.program_id(0)
    praw = ptr_smem[0]
    pc = jnp.clip(praw, 0, QUEUE - BATCH)
    g = i * BLK
    in_keys = jnp.logical_and(g >= pc, g < pc + BATCH)

    @pl.when(in_keys)
    def _():
        out[...] = keys[...]

    @pl.when(jnp.logical_not(in_keys))
    def _():
        out[...] = data[...]

    ptr_out[0] = (praw + BATCH) % QUEUE


def kernel(keys, data, ptr):
    grid_spec = pltpu.PrefetchScalarGridSpec(
        num_scalar_prefetch=1,
        grid=(NCH,),
        in_specs=[
            pl.BlockSpec((BATCH, FEAT), lambda i, p: (0, 0)),
            pl.BlockSpec(
                (BLK, FEAT),
                lambda i, p: (
                    jnp.where(
                        jnp.logical_and(
                            i * BLK >= jnp.clip(p[0], 0, QUEUE - BATCH),
                            i * BLK < jnp.clip(p[0], 0, QUEUE - BATCH) + BATCH,
                        ),
                        jnp.minimum(i + 1, NCH - 1),
                        i,
                    ),
                    0,
                ),
            ),
        ],
        out_specs=[
            pl.BlockSpec((BLK, FEAT), lambda i, p: (i, 0)),
            pl.BlockSpec(memory_space=pltpu.SMEM),
        ],
    )
    out, new_ptr = pl.pallas_call(
        _queue_body,
        grid_spec=grid_spec,
        out_shape=(
            jax.ShapeDtypeStruct((QUEUE, FEAT), jnp.float32),
            jax.ShapeDtypeStruct((1,), jnp.int32),
        ),
    )(ptr, keys, data)
    return out, new_ptr


# speedup vs baseline: 41.2852x; 1.0714x over previous
import jax
import jax.numpy as jnp
from jax.experimental import pallas as pl
from jax.experimental.pallas import tpu as pltpu

QUEUE = 65536
FEAT = 128
BATCH = 4096
BLK = 8192
NCH = QUEUE // BLK


def _queue_body(ptr_smem, keys, data, out, ptr_out):
    i = pl.program_id(0)
    praw = ptr_smem[0]
    pc = jnp.clip(praw, 0, QUEUE - BATCH)
    g = i * BLK
    has_window = jnp.logical_and(pc >= g, pc < g + BLK)

    out[...] = data[...]

    @pl.when(has_window)
    def _():
        out[pl.ds(pl.multiple_of(pc - g, 8), BATCH), :] = keys[...]

    ptr_out[0] = (praw + BATCH) % QUEUE


def kernel(keys, data, ptr):
    grid_spec = pltpu.PrefetchScalarGridSpec(
        num_scalar_prefetch=1,
        grid=(NCH,),
        in_specs=[
            pl.BlockSpec((BATCH, FEAT), lambda i, p: (0, 0)),
            pl.BlockSpec((BLK, FEAT), lambda i, p: (i, 0)),
        ],
        out_specs=[
            pl.BlockSpec((BLK, FEAT), lambda i, p: (i, 0)),
            pl.BlockSpec(memory_space=pltpu.SMEM),
        ],
    )
    out, new_ptr = pl.pallas_call(
        _queue_body,
        grid_spec=grid_spec,
        out_shape=(
            jax.ShapeDtypeStruct((QUEUE, FEAT), jnp.float32),
            jax.ShapeDtypeStruct((1,), jnp.int32),
        ),
    )(ptr, keys, data)
    return out, new_ptr


# speedup vs baseline: 43.9400x; 1.0643x over previous
import jax
import jax.numpy as jnp
from jax.experimental import pallas as pl
from jax.experimental.pallas import tpu as pltpu

QUEUE = 65536
FEAT = 128
BATCH = 4096
BLK = 16384
NCH = QUEUE // BLK


def _queue_body(ptr_smem, keys, data, out, ptr_out):
    i = pl.program_id(0)
    praw = ptr_smem[0]
    pc = jnp.clip(praw, 0, QUEUE - BATCH)
    g = i * BLK
    has_window = jnp.logical_and(pc >= g, pc < g + BLK)

    out[...] = data[...]

    @pl.when(has_window)
    def _():
        out[pl.ds(pl.multiple_of(pc - g, 8), BATCH), :] = keys[...]

    ptr_out[0] = (praw + BATCH) % QUEUE


def kernel(keys, data, ptr):
    grid_spec = pltpu.PrefetchScalarGridSpec(
        num_scalar_prefetch=1,
        grid=(NCH,),
        in_specs=[
            pl.BlockSpec((BATCH, FEAT), lambda i, p: (0, 0)),
            pl.BlockSpec((BLK, FEAT), lambda i, p: (i, 0)),
        ],
        out_specs=[
            pl.BlockSpec((BLK, FEAT), lambda i, p: (i, 0)),
            pl.BlockSpec(memory_space=pltpu.SMEM),
        ],
    )
    out, new_ptr = pl.pallas_call(
        _queue_body,
        grid_spec=grid_spec,
        out_shape=(
            jax.ShapeDtypeStruct((QUEUE, FEAT), jnp.float32),
            jax.ShapeDtypeStruct((1,), jnp.int32),
        ),
    )(ptr, keys, data)
    return out, new_ptr
